# Initial kernel scaffold; baseline (speedup 1.0000x reference)
import jax
import jax.numpy as jnp
from jax.experimental import pallas as pl


def kernel(x, params):
    raise NotImplementedError("write your pallas kernel here")


# speedup vs baseline: 5.0532x; 5.0532x over previous
import functools

import jax
import jax.numpy as jnp
from jax import lax
from jax.experimental import pallas as pl
from jax.experimental.pallas import tpu as pltpu
from jax.experimental.pallas import tpu_sc as plsc

B = 4
N = 2048
K = 20
KPAD = 128
CP = 128
ROWS = B * N
BLK = 256
NBLK = ROWS // BLK
EPS = 1e-5

NC = 2
NS = 16
NW = NC * NS
PPW = ROWS // NW
CHP = 8
CHROWS = CHP * K
NCH = PPW // CHP
IDXCH = 64


def _lrelu(v):
    return jnp.where(v >= 0.0, v, 0.2 * v)


def _bf(v):
    return v.astype(jnp.bfloat16)


def _knn_body(blk_ref, slab_ref, xx_ref, idx_ref, dist_ref):
    b = pl.program_id(0)
    blk = blk_ref[...]
    slab = slab_ref[...]
    inner = lax.dot_general(_bf(blk), _bf(slab), (((1,), (1,)), ((), ())),
                            preferred_element_type=jnp.float32)
    xxb = jnp.sum(blk * blk, axis=1, keepdims=True)
    xxs = xx_ref[0]
    dist_ref[...] = (xxb + xxs) - 2.0 * inner
    iota = lax.broadcasted_iota(jnp.int32, (BLK, N), 1)
    base = b * N
    cols = []
    for _ in range(K):
        d = dist_ref[...]
        rowmin = jnp.min(d, axis=1, keepdims=True)
        cand = jnp.where(d == rowmin, iota, jnp.int32(2 * N))
        sel = jnp.min(cand, axis=1, keepdims=True)
        cols.append(sel + base)
        dist_ref[...] = jnp.where(iota == sel, jnp.float32(jnp.inf), d)
    cols.append(jnp.zeros((BLK, KPAD - K), jnp.int32))
    idx_ref[...] = jnp.concatenate(cols, axis=1)


def _knn(xT, xxrow):
    return pl.pallas_call(
        _knn_body,
        grid=(B, N // BLK),
        in_specs=[
            pl.BlockSpec((BLK, CP), lambda b, i: (b * (N // BLK) + i, 0)),
            pl.BlockSpec((N, CP), lambda b, i: (b, 0)),
            pl.BlockSpec((1, 1, N), lambda b, i: (b, 0, 0)),
        ],
        out_specs=pl.BlockSpec((BLK, KPAD), lambda b, i: (b * (N // BLK) + i, 0)),
        out_shape=jax.ShapeDtypeStruct((ROWS, KPAD), jnp.int32),
        scratch_shapes=[pltpu.VMEM((BLK, N), jnp.float32)],
    )(xT, xT, xxrow)


def _sc_body(ce, x_hbm, idx_hbm, e_hbm,
             idx_v, lst_v, buf_v, xi_v, e_v, sem0, sem1, osem0, osem1):
    wid = lax.axis_index("s") * NC + lax.axis_index("c")
    base = wid * PPW

    lane = lax.iota(jnp.int32, NS)

    for t in range(PPW // IDXCH):
        pltpu.sync_copy(idx_hbm.at[pl.ds(base + t * IDXCH, IDXCH)], idx_v)

        def build(pp, carry, t=t):
            for half in range(2):
                v = idx_v[pp, pl.ds(half * NS, NS)]
                pos = (t * IDXCH + pp) * K + half * NS + lane
                msk = (half * NS + lane) < K
                plsc.store_scatter(lst_v, (pos,), v, mask=msk)
            return carry

        lax.fori_loop(0, IDXCH, build, 0)

    sems = (sem0, sem1)
    osems = (osem0, osem1)
    ew = K * ce

    def fire(g, slot):
        pltpu.async_copy(
            x_hbm.at[lst_v.at[pl.ds(g * CHROWS, CHROWS)]],
            buf_v.at[slot], sems[slot])

    fire(0, 0)
    fire(1, 1)

    def chunk(g, slot):
        pltpu.make_async_copy(
            x_hbm.at[lst_v.at[pl.ds(g * CHROWS, CHROWS)]],
            buf_v.at[slot], sems[slot]).wait()
        pltpu.sync_copy(x_hbm.at[pl.ds(base + g * CHP, CHP)], xi_v)

        @pl.when(g >= 2)
        def _():
            pltpu.make_async_copy(e_v.at[slot],
                                  e_hbm.at[pl.ds(base + (g - 2) * CHP, CHP)],
                                  osems[slot]).wait()

        def point(p, carry):
            row = p * K
            for c in range(ce // NS):
                cs = pl.ds(c * NS, NS)
                xi = xi_v[p, cs]
                for j in range(K):
                    e_v[slot, p, pl.ds(j * 2 * ce + c * NS, NS)] = xi
                    e_v[slot, p, pl.ds(j * 2 * ce + ce + c * NS, NS)] = (
                        buf_v[slot, row + j, cs] - xi)
            return carry

        lax.fori_loop(0, CHP, point, 0)

        pltpu.async_copy(e_v.at[slot],
                         e_hbm.at[pl.ds(base + g * CHP, CHP)], osems[slot])

        @pl.when(g + 2 < NCH)
        def _():
            fire(g + 2, slot)

    def pair(t, carry):
        chunk(2 * t, 0)
        chunk(2 * t + 1, 1)
        return carry

    lax.fori_loop(0, NCH // 2, pair, 0)

    pltpu.make_async_copy(e_v.at[0],
                          e_hbm.at[pl.ds(base + (NCH - 2) * CHP, CHP)],
                          osems[0]).wait()
    pltpu.make_async_copy(e_v.at[1],
                          e_hbm.at[pl.ds(base + (NCH - 1) * CHP, CHP)],
                          osems[1]).wait()


def _sc_edges(xT, idx, ce):
    mesh = plsc.VectorSubcoreMesh(core_axis_name="c", subcore_axis_name="s",
                                  num_cores=NC, num_subcores=NS)
    fn = pl.kernel(
        functools.partial(_sc_body, ce),
        out_type=jax.ShapeDtypeStruct((ROWS, K * 2 * ce), jnp.float32),
        mesh=mesh,
        compiler_params=pltpu.CompilerParams(needs_layout_passes=False),
        scratch_types=[
            pltpu.VMEM((IDXCH, KPAD), jnp.int32),
            pltpu.VMEM((PPW * K,), jnp.int32),
            pltpu.VMEM((2, CHROWS, CP), jnp.float32),
            pltpu.VMEM((CHP, CP), jnp.float32),
            pltpu.VMEM((2, CHP, K * 2 * ce), jnp.float32),
            pltpu.SemaphoreType.DMA,
            pltpu.SemaphoreType.DMA,
            pltpu.SemaphoreType.DMA,
            pltpu.SemaphoreType.DMA,
        ],
    )
    return fn(xT, idx)


def _ered_body(w2c, e_ref, wt_ref, m_ref, sh_ref, sq_ref):
    w = _bf(wt_ref[...])
    e = _bf(e_ref[...])
    hj = jnp.dot(e[:, :w2c], w, preferred_element_type=jnp.float32)
    mx = hj
    sm = hj
    sq = hj * hj
    for j in range(1, K):
        hj = jnp.dot(e[:, j * w2c:(j + 1) * w2c], w,
                     preferred_element_type=jnp.float32)
        mx = jnp.maximum(mx, hj)
        sm = sm + hj
        sq = sq + hj * hj
    m_ref[...] = mx
    sh_ref[...] = jnp.sum(sm, axis=0, keepdims=True)[None]
    sq_ref[...] = jnp.sum(sq, axis=0, keepdims=True)[None]


def _edgered(E, wt):
    w2c = wt.shape[0]
    o = wt.shape[1]
    return pl.pallas_call(
        functools.partial(_ered_body, w2c),
        grid=(NBLK,),
        in_specs=[
            pl.BlockSpec((BLK, K * w2c), lambda i: (i, 0)),
            pl.BlockSpec((w2c, o), lambda i: (0, 0)),
        ],
        out_specs=[
            pl.BlockSpec((BLK, o), lambda i: (i, 0)),
            pl.BlockSpec((1, 1, o), lambda i: (i, 0, 0)),
            pl.BlockSpec((1, 1, o), lambda i: (i, 0, 0)),
        ],
        out_shape=[
            jax.ShapeDtypeStruct((ROWS, o), jnp.float32),
            jax.ShapeDtypeStruct((NBLK, 1, o), jnp.float32),
            jax.ShapeDtypeStruct((NBLK, 1, o), jnp.float32),
        ],
    )(E, wt)


def _pstats_body(cnt, sp_ref, qp_ref, mu_ref, den_ref):
    mu = jnp.sum(sp_ref[:, 0, :], axis=0, keepdims=True) / cnt
    var = jnp.sum(qp_ref[:, 0, :], axis=0, keepdims=True) / cnt - mu * mu
    mu_ref[...] = mu
    den_ref[...] = jnp.sqrt(var + EPS)


def _pstats(sp, qp, cnt):
    c = sp.shape[2]
    return pl.pallas_call(
        functools.partial(_pstats_body, float(cnt)),
        in_specs=[pl.BlockSpec((NBLK, 1, c), lambda: (0, 0, 0)),
                  pl.BlockSpec((NBLK, 1, c), lambda: (0, 0, 0))],
        out_specs=[pl.BlockSpec((1, c), lambda: (0, 0)),
                   pl.BlockSpec((1, c), lambda: (0, 0))],
        out_shape=[jax.ShapeDtypeStruct((1, c), jnp.float32),
                   jax.ShapeDtypeStruct((1, c), jnp.float32)],
    )(sp, qp)


def _bn_apply(v, g, b, mu, den):
    return g * (v - mu) / den + b


def _apply_body(m_ref, g_ref, b_ref, mu_ref, den_ref, x_ref, xsq_ref):
    v = _lrelu(_bn_apply(m_ref[...], g_ref[...], b_ref[...],
                         mu_ref[...], den_ref[...]))
    x_ref[...] = jnp.concatenate(
        [v, jnp.zeros((BLK, CP - v.shape[1]), jnp.float32)], axis=1)
    xsq_ref[...] = jnp.sum(v * v, axis=1, keepdims=True)


def _apply(m, g, b, mu, den):
    c = m.shape[1]
    return pl.pallas_call(
        _apply_body,
        grid=(NBLK,),
        in_specs=[
            pl.BlockSpec((BLK, c), lambda i: (i, 0)),
            pl.BlockSpec((1, c), lambda i: (0, 0)),
            pl.BlockSpec((1, c), lambda i: (0, 0)),
            pl.BlockSpec((1, c), lambda i: (0, 0)),
            pl.BlockSpec((1, c), lambda i: (0, 0)),
        ],
        out_specs=[
            pl.BlockSpec((BLK, CP), lambda i: (i, 0)),
            pl.BlockSpec((BLK, 1), lambda i: (i, 0)),
        ],
        out_shape=[
            jax.ShapeDtypeStruct((ROWS, CP), jnp.float32),
            jax.ShapeDtypeStruct((ROWS, 1), jnp.float32),
        ],
    )(m, g.reshape(1, c), b.reshape(1, c), mu, den)


def _conv_body(x1_ref, x2_ref, x3_ref, m4_ref, g4_ref, b4_ref, mu_ref,
               den_ref, w_ref, x4_ref, sg_ref, sq_ref, gm_ref):
    x4 = _lrelu(_bn_apply(m4_ref[...], g4_ref[...], b4_ref[...],
                          mu_ref[...], den_ref[...]))
    x4_ref[...] = x4
    cat = jnp.concatenate([x1_ref[...][:, :64], x2_ref[...][:, :64],
                           x3_ref[...][:, :64], x4], axis=1)
    g = jnp.dot(_bf(cat), _bf(w_ref[...]), preferred_element_type=jnp.float32)
    sg_ref[...] = jnp.sum(g, axis=0, keepdims=True)[None]
    sq_ref[...] = jnp.sum(g * g, axis=0, keepdims=True)[None]
    gm_ref[...] = jnp.max(g, axis=0, keepdims=True)[None, None]


def _conv_head(x1T, x2T, x3T, m4, g4, b4, mu4, den4, convWT):
    return pl.pallas_call(
        _conv_body,
        grid=(NBLK,),
        in_specs=[
            pl.BlockSpec((BLK, CP), lambda i: (i, 0)),
            pl.BlockSpec((BLK, CP), lambda i: (i, 0)),
            pl.BlockSpec((BLK, CP), lambda i: (i, 0)),
            pl.BlockSpec((BLK, 128), lambda i: (i, 0)),
            pl.BlockSpec((1, 128), lambda i: (0, 0)),
            pl.BlockSpec((1, 128), lambda i: (0, 0)),
            pl.BlockSpec((1, 128), lambda i: (0, 0)),
            pl.BlockSpec((1, 128), lambda i: (0, 0)),
            pl.BlockSpec((320, 1024), lambda i: (0, 0)),
        ],
        out_specs=[
            pl.BlockSpec((BLK, 128), lambda i: (i, 0)),
            pl.BlockSpec((1, 1, 1024), lambda i: (i, 0, 0)),
            pl.BlockSpec((1, 1, 1024), lambda i: (i, 0, 0)),
            pl.BlockSpec((1, 1, 1, 1024), lambda i: (i // 8, i % 8, 0, 0)),
        ],
        out_shape=[
            jax.ShapeDtypeStruct((ROWS, 128), jnp.float32),
            jax.ShapeDtypeStruct((NBLK, 1, 1024), jnp.float32),
            jax.ShapeDtypeStruct((NBLK, 1, 1024), jnp.float32),
            jax.ShapeDtypeStruct((B, 8, 1, 1024), jnp.float32),
        ],
    )(x1T, x2T, x3T, m4, g4.reshape(1, 128), b4.reshape(1, 128),
      mu4, den4, convWT)


def _gmax_body(sg_ref, sq_ref, gm_ref, g_ref, b_ref, w2t_ref, vb_ref):
    cnt = jnp.float32(ROWS)
    mu = jnp.sum(sg_ref[:, 0, :], axis=0, keepdims=True) / cnt
    var = jnp.sum(sq_ref[:, 0, :], axis=0, keepdims=True) / cnt - mu * mu
    den = jnp.sqrt(var + EPS)
    rows = [jnp.max(gm_ref[b, :, 0, :], axis=0, keepdims=True)
            for b in range(B)]
    gm = jnp.concatenate(rows, axis=0)
    feat = _lrelu(_bn_apply(gm, g_ref[...], b_ref[...], mu, den))
    vb = jnp.dot(_bf(feat), _bf(w2t_ref[...]),
                 preferred_element_type=jnp.float32)
    vb_ref[...] = vb[:, None, :]


def _gmax_head(sg, sq, gmax, conv_g, conv_b, seg1W2T):
    return pl.pallas_call(
        _gmax_body,
        in_specs=[pl.BlockSpec((NBLK, 1, 1024), lambda: (0, 0, 0)),
                  pl.BlockSpec((NBLK, 1, 1024), lambda: (0, 0, 0)),
                  pl.BlockSpec((B, 8, 1, 1024), lambda: (0, 0, 0, 0)),
                  pl.BlockSpec((1, 1024), lambda: (0, 0)),
                  pl.BlockSpec((1, 1024), lambda: (0, 0)),
                  pl.BlockSpec((1024, 512), lambda: (0, 0))],
        out_specs=pl.BlockSpec((B, 1, 512), lambda: (0, 0, 0)),
        out_shape=jax.ShapeDtypeStruct((B, 1, 512), jnp.float32),
    )(sg, sq, gmax, conv_g.reshape(1, 1024), conv_b.reshape(1, 1024), seg1W2T)


def _seg1_body(x1_ref, x2_ref, x3_ref, x4_ref, w_ref, vb_ref,
               h_ref, sp_ref, qp_ref):
    cat = jnp.concatenate([x1_ref[...][:, :64], x2_ref[...][:, :64],
                           x3_ref[...][:, :64], x4_ref[...]], axis=1)
    h = jnp.dot(_bf(cat), _bf(w_ref[...]),
                preferred_element_type=jnp.float32) + vb_ref[0]
    h_ref[...] = h
    sp_ref[...] = jnp.sum(h, axis=0, keepdims=True)[None]
    qp_ref[...] = jnp.sum(h * h, axis=0, keepdims=True)[None]


def _seg1(x1T, x2T, x3T, x4T, seg1W1T, vbias):
    return pl.pallas_call(
        _seg1_body,
        grid=(NBLK,),
        in_specs=[
            pl.BlockSpec((BLK, CP), lambda i: (i, 0)),
            pl.BlockSpec((BLK, CP), lambda i: (i, 0)),
            pl.BlockSpec((BLK, CP), lambda i: (i, 0)),
            pl.BlockSpec((BLK, 128), lambda i: (i, 0)),
            pl.BlockSpec((320, 512), lambda i: (0, 0)),
            pl.BlockSpec((1, 1, 512), lambda i: (i // 8, 0, 0)),
        ],
        out_specs=[
            pl.BlockSpec((BLK, 512), lambda i: (i, 0)),
            pl.BlockSpec((1, 1, 512), lambda i: (i, 0, 0)),
            pl.BlockSpec((1, 1, 512), lambda i: (i, 0, 0)),
        ],
        out_shape=[
            jax.ShapeDtypeStruct((ROWS, 512), jnp.float32),
            jax.ShapeDtypeStruct((NBLK, 1, 512), jnp.float32),
            jax.ShapeDtypeStruct((NBLK, 1, 512), jnp.float32),
        ],
    )(x1T, x2T, x3T, x4T, seg1W1T, vbias)


def _seg_body(h_ref, g_ref, b_ref, mu_ref, den_ref, w_ref, bias_ref,
              o_ref, sp_ref, qp_ref):
    t = _lrelu(_bn_apply(h_ref[...], g_ref[...], b_ref[...],
                         mu_ref[...], den_ref[...]))
    o = jnp.dot(_bf(t), _bf(w_ref[...]),
                preferred_element_type=jnp.float32) + bias_ref[...]
    o_ref[...] = o
    if sp_ref is not None:
        sp_ref[...] = jnp.sum(o, axis=0, keepdims=True)[None]
        qp_ref[...] = jnp.sum(o * o, axis=0, keepdims=True)[None]


def _seg(h, g, b, mu, den, wT, bias, with_stats):
    cin = h.shape[1]
    cout = wT.shape[1]
    body = _seg_body if with_stats else (
        lambda h_ref, g_ref, b_ref, mu_ref, den_ref, w_ref, bias_ref, o_ref:
        _seg_body(h_ref, g_ref, b_ref, mu_ref, den_ref, w_ref, bias_ref,
                  o_ref, None, None))
    out_specs = [pl.BlockSpec((BLK, cout), lambda i: (i, 0))]
    out_shape = [jax.ShapeDtypeStruct((ROWS, cout), jnp.float32)]
    if with_stats:
        out_specs += [pl.BlockSpec((1, 1, cout), lambda i: (i, 0, 0)),
                      pl.BlockSpec((1, 1, cout), lambda i: (i, 0, 0))]
        out_shape += [jax.ShapeDtypeStruct((NBLK, 1, cout), jnp.float32),
                      jax.ShapeDtypeStruct((NBLK, 1, cout), jnp.float32)]
    return pl.pallas_call(
        body,
        grid=(NBLK,),
        in_specs=[
            pl.BlockSpec((BLK, cin), lambda i: (i, 0)),
            pl.BlockSpec((1, cin), lambda i: (0, 0)),
            pl.BlockSpec((1, cin), lambda i: (0, 0)),
            pl.BlockSpec((1, cin), lambda i: (0, 0)),
            pl.BlockSpec((1, cin), lambda i: (0, 0)),
            pl.BlockSpec((cin, cout), lambda i: (0, 0)),
            pl.BlockSpec((1, cout), lambda i: (0, 0)),
        ],
        out_specs=out_specs,
        out_shape=out_shape,
    )(h, g.reshape(1, cin), b.reshape(1, cin), mu, den, wT, bias)


def _shadow_stats(E, w, ce, cin):
    e4 = E.reshape(B, N, K, 2 * ce)
    if ce != cin:
        e4 = jnp.concatenate([e4[..., :cin], e4[..., ce:ce + cin]], axis=-1)
    edge = jax.lax.optimization_barrier(jnp.transpose(e4, (0, 3, 1, 2)))
    h = jnp.einsum('oc,bcnk->bonk', w, edge)
    mu = jnp.mean(h, axis=(0, 2, 3), keepdims=True)
    var = jnp.mean((h - mu) ** 2, axis=(0, 2, 3), keepdims=True)
    o = w.shape[0]
    return mu.reshape(1, o), jnp.sqrt(var + EPS).reshape(1, o)


def _edge_layer(xTp, xxrow, w, cin, ce):
    w1t = w[:, :cin].T
    w2t = w[:, cin:].T
    if ce != cin:
        w1t = jnp.pad(w1t, ((0, ce - cin), (0, 0)))
        w2t = jnp.pad(w2t, ((0, ce - cin), (0, 0)))
    wt = jnp.concatenate([w1t, w2t], axis=0)
    idx = _knn(xTp, xxrow)
    E = _sc_edges(xTp, idx, ce)
    m, _sh_p, _sq_p = _edgered(E, wt)
    mu, den = _shadow_stats(E, w, ce, cin)
    return m, mu, den


def kernel(x, params):
    p = params
    x0 = x.reshape(ROWS, 3)
    x0p = jnp.pad(x0, ((0, 0), (0, CP - 3)))
    xx1 = jnp.sum(jnp.transpose(x, (0, 2, 1)) ** 2, axis=1, keepdims=True)

    m1, mu1, den1 = _edge_layer(x0p, xx1, p['ec1_W'], 3, 16)
    x1p, xsq1 = _apply(m1, p['ec1_g'], p['ec1_b'], mu1, den1)

    m2, mu2, den2 = _edge_layer(x1p, xsq1.reshape(B, 1, N), p['ec2_W'], 64, 64)
    x2p, xsq2 = _apply(m2, p['ec2_g'], p['ec2_b'], mu2, den2)

    m3, mu3, den3 = _edge_layer(x2p, xsq2.reshape(B, 1, N), p['ec3_W'], 64, 64)
    x3p, xsq3 = _apply(m3, p['ec3_g'], p['ec3_b'], mu3, den3)

    m4, mu4, den4 = _edge_layer(x3p, xsq3.reshape(B, 1, N), p['ec4_W'], 64, 64)

    x4T, sg, sq, gmax = _conv_head(x1p, x2p, x3p, m4, p['ec4_g'], p['ec4_b'],
                                   mu4, den4, p['conv_W'].T)
    vbias = _gmax_head(sg, sq, gmax, p['conv_g'], p['conv_b'],
                       p['seg1_W'][:, 320:].T)
    h1, sp1, qp1 = _seg1(x1p, x2p, x3p, x4T, p['seg1_W'][:, :320].T, vbias)
    mu_s1, den_s1 = _pstats(sp1, qp1, ROWS)
    zb256 = jnp.zeros((1, 256), jnp.float32)
    h2, sp2, qp2 = _seg(h1, p['seg1_g'], p['seg1_b'], mu_s1, den_s1,
                        p['seg2_W'].T, zb256, True)
    mu_s2, den_s2 = _pstats(sp2, qp2, ROWS)
    w3t = jnp.pad(p['seg3_W'].T, ((0, 0), (0, 14)))
    b3 = jnp.pad(p['seg3_bias'].reshape(1, 50), ((0, 0), (0, 14)))
    (out,) = _seg(h2, p['seg2_g'], p['seg2_b'], mu_s2, den_s2, w3t, b3, False)
    return out[:, :50].reshape(B, N, 50)


# speedup vs baseline: 5.3380x; 1.0564x over previous
import functools

import jax
import jax.numpy as jnp
from jax import lax
from jax.experimental import pallas as pl
from jax.experimental.pallas import tpu as pltpu
from jax.experimental.pallas import tpu_sc as plsc

B = 4
N = 2048
K = 20
KPAD = 128
CP = 128
ROWS = B * N
BLK = 256
NBLK = ROWS // BLK
EPS = 1e-5

NC = 2
NS = 16
NW = NC * NS
PPW = ROWS // NW
CHP = 8
CHROWS = CHP * K
NCH = PPW // CHP
IDXCH = 64


def _lrelu(v):
    return jnp.where(v >= 0.0, v, 0.2 * v)


def _bf(v):
    return v.astype(jnp.bfloat16)


KBLK = 512


def _knn_body(blk_ref, slab_ref, xx_ref, idx_ref, dist_ref):
    b = pl.program_id(0)
    blk = blk_ref[...]
    slab = slab_ref[...]
    inner = lax.dot_general(_bf(blk), _bf(slab), (((1,), (1,)), ((), ())),
                            preferred_element_type=jnp.float32)
    xxb = jnp.sum(blk * blk, axis=1, keepdims=True)
    xxs = xx_ref[0]
    dist_ref[...] = (xxb + xxs) - 2.0 * inner
    iota = lax.broadcasted_iota(jnp.int32, (KBLK, N), 1)
    base = b * N
    cols = []
    for _ in range(K):
        d = dist_ref[...]
        rowmin = jnp.min(d, axis=1, keepdims=True)
        cand = jnp.where(d == rowmin, iota, jnp.int32(2 * N))
        sel = jnp.min(cand, axis=1, keepdims=True)
        cols.append(sel + base)
        dist_ref[...] = jnp.where(iota == sel, jnp.float32(jnp.inf), d)
    cols.append(jnp.zeros((KBLK, KPAD - K), jnp.int32))
    idx_ref[...] = jnp.concatenate(cols, axis=1)


def _knn(xT, xxrow):
    return pl.pallas_call(
        _knn_body,
        grid=(B, N // KBLK),
        in_specs=[
            pl.BlockSpec((KBLK, CP), lambda b, i: (b * (N // KBLK) + i, 0)),
            pl.BlockSpec((N, CP), lambda b, i: (b, 0)),
            pl.BlockSpec((1, 1, N), lambda b, i: (b, 0, 0)),
        ],
        out_specs=pl.BlockSpec((KBLK, KPAD),
                               lambda b, i: (b * (N // KBLK) + i, 0)),
        out_shape=jax.ShapeDtypeStruct((ROWS, KPAD), jnp.int32),
        scratch_shapes=[pltpu.VMEM((KBLK, N), jnp.float32)],
    )(xT, xT, xxrow)


def _sc_body(ce, x_hbm, idx_hbm, e_hbm,
             idx_v, lst_v, buf_v, xi_v, e_v, sem0, sem1, osem0, osem1):
    wid = lax.axis_index("s") * NC + lax.axis_index("c")
    base = wid * PPW

    lane = lax.iota(jnp.int32, NS)

    for t in range(PPW // IDXCH):
        pltpu.sync_copy(idx_hbm.at[pl.ds(base + t * IDXCH, IDXCH)], idx_v)

        def build(pp, carry, t=t):
            for half in range(2):
                v = idx_v[pp, pl.ds(half * NS, NS)]
                pos = (t * IDXCH + pp) * K + half * NS + lane
                msk = (half * NS + lane) < K
                plsc.store_scatter(lst_v, (pos,), v, mask=msk)
            return carry

        lax.fori_loop(0, IDXCH, build, 0)

    sems = (sem0, sem1)
    osems = (osem0, osem1)
    ew = K * ce

    def fire(g, slot):
        pltpu.async_copy(
            x_hbm.at[lst_v.at[pl.ds(g * CHROWS, CHROWS)]],
            buf_v.at[slot], sems[slot])

    fire(0, 0)
    fire(1, 1)

    def chunk(g, slot):
        pltpu.make_async_copy(
            x_hbm.at[lst_v.at[pl.ds(g * CHROWS, CHROWS)]],
            buf_v.at[slot], sems[slot]).wait()
        pltpu.sync_copy(x_hbm.at[pl.ds(base + g * CHP, CHP)], xi_v)

        @pl.when(g >= 2)
        def _():
            pltpu.make_async_copy(e_v.at[slot],
                                  e_hbm.at[pl.ds(base + (g - 2) * CHP, CHP)],
                                  osems[slot]).wait()

        def point(p, carry):
            row = p * K
            for c in range(ce // NS):
                cs = pl.ds(c * NS, NS)
                xi = xi_v[p, cs]
                for j in range(K):
                    e_v[slot, p, pl.ds(j * 2 * ce + c * NS, NS)] = xi
                    e_v[slot, p, pl.ds(j * 2 * ce + ce + c * NS, NS)] = (
                        buf_v[slot, row + j, cs] - xi)
            return carry

        lax.fori_loop(0, CHP, point, 0)

        pltpu.async_copy(e_v.at[slot],
                         e_hbm.at[pl.ds(base + g * CHP, CHP)], osems[slot])

        @pl.when(g + 2 < NCH)
        def _():
            fire(g + 2, slot)

    def pair(t, carry):
        chunk(2 * t, 0)
        chunk(2 * t + 1, 1)
        return carry

    lax.fori_loop(0, NCH // 2, pair, 0)

    pltpu.make_async_copy(e_v.at[0],
                          e_hbm.at[pl.ds(base + (NCH - 2) * CHP, CHP)],
                          osems[0]).wait()
    pltpu.make_async_copy(e_v.at[1],
                          e_hbm.at[pl.ds(base + (NCH - 1) * CHP, CHP)],
                          osems[1]).wait()


def _sc_edges(xT, idx, ce):
    mesh = plsc.VectorSubcoreMesh(core_axis_name="c", subcore_axis_name="s",
                                  num_cores=NC, num_subcores=NS)
    fn = pl.kernel(
        functools.partial(_sc_body, ce),
        out_type=jax.ShapeDtypeStruct((ROWS, K * 2 * ce), jnp.float32),
        mesh=mesh,
        compiler_params=pltpu.CompilerParams(needs_layout_passes=False),
        scratch_types=[
            pltpu.VMEM((IDXCH, KPAD), jnp.int32),
            pltpu.VMEM((PPW * K,), jnp.int32),
            pltpu.VMEM((2, CHROWS, CP), jnp.float32),
            pltpu.VMEM((CHP, CP), jnp.float32),
            pltpu.VMEM((2, CHP, K * 2 * ce), jnp.float32),
            pltpu.SemaphoreType.DMA,
            pltpu.SemaphoreType.DMA,
            pltpu.SemaphoreType.DMA,
            pltpu.SemaphoreType.DMA,
        ],
    )
    return fn(xT, idx)


def _ered_body(w2c, e_ref, wt_ref, m_ref):
    w = _bf(wt_ref[...])
    e = _bf(e_ref[...])
    mx = jnp.dot(e[:, :w2c], w, preferred_element_type=jnp.float32)
    for j in range(1, K):
        hj = jnp.dot(e[:, j * w2c:(j + 1) * w2c], w,
                     preferred_element_type=jnp.float32)
        mx = jnp.maximum(mx, hj)
    m_ref[...] = mx


def _edgered(E, wt):
    w2c = wt.shape[0]
    o = wt.shape[1]
    return pl.pallas_call(
        functools.partial(_ered_body, w2c),
        grid=(NBLK,),
        in_specs=[
            pl.BlockSpec((BLK, K * w2c), lambda i: (i, 0)),
            pl.BlockSpec((w2c, o), lambda i: (0, 0)),
        ],
        out_specs=pl.BlockSpec((BLK, o), lambda i: (i, 0)),
        out_shape=jax.ShapeDtypeStruct((ROWS, o), jnp.float32),
    )(E, wt)


def _pstats_body(cnt, sp_ref, qp_ref, mu_ref, den_ref):
    mu = jnp.sum(sp_ref[:, 0, :], axis=0, keepdims=True) / cnt
    var = jnp.sum(qp_ref[:, 0, :], axis=0, keepdims=True) / cnt - mu * mu
    mu_ref[...] = mu
    den_ref[...] = jnp.sqrt(var + EPS)


def _pstats(sp, qp, cnt):
    c = sp.shape[2]
    return pl.pallas_call(
        functools.partial(_pstats_body, float(cnt)),
        in_specs=[pl.BlockSpec((NBLK, 1, c), lambda: (0, 0, 0)),
                  pl.BlockSpec((NBLK, 1, c), lambda: (0, 0, 0))],
        out_specs=[pl.BlockSpec((1, c), lambda: (0, 0)),
                   pl.BlockSpec((1, c), lambda: (0, 0))],
        out_shape=[jax.ShapeDtypeStruct((1, c), jnp.float32),
                   jax.ShapeDtypeStruct((1, c), jnp.float32)],
    )(sp, qp)


def _bn_apply(v, g, b, mu, den):
    return g * (v - mu) / den + b


def _apply_body(m_ref, g_ref, b_ref, mu_ref, den_ref, x_ref, xsq_ref):
    v = _lrelu(_bn_apply(m_ref[...], g_ref[...], b_ref[...],
                         mu_ref[...], den_ref[...]))
    x_ref[...] = jnp.concatenate(
        [v, jnp.zeros((BLK, CP - v.shape[1]), jnp.float32)], axis=1)
    xsq_ref[...] = jnp.sum(v * v, axis=1, keepdims=True)


def _apply(m, g, b, mu, den):
    c = m.shape[1]
    return pl.pallas_call(
        _apply_body,
        grid=(NBLK,),
        in_specs=[
            pl.BlockSpec((BLK, c), lambda i: (i, 0)),
            pl.BlockSpec((1, c), lambda i: (0, 0)),
            pl.BlockSpec((1, c), lambda i: (0, 0)),
            pl.BlockSpec((1, c), lambda i: (0, 0)),
            pl.BlockSpec((1, c), lambda i: (0, 0)),
        ],
        out_specs=[
            pl.BlockSpec((BLK, CP), lambda i: (i, 0)),
            pl.BlockSpec((BLK, 1), lambda i: (i, 0)),
        ],
        out_shape=[
            jax.ShapeDtypeStruct((ROWS, CP), jnp.float32),
            jax.ShapeDtypeStruct((ROWS, 1), jnp.float32),
        ],
    )(m, g.reshape(1, c), b.reshape(1, c), mu, den)


def _conv_body(x1_ref, x2_ref, x3_ref, m4_ref, g4_ref, b4_ref, mu_ref,
               den_ref, w_ref, x4_ref, sg_ref, sq_ref, gm_ref):
    x4 = _lrelu(_bn_apply(m4_ref[...], g4_ref[...], b4_ref[...],
                          mu_ref[...], den_ref[...]))
    x4_ref[...] = x4
    cat = jnp.concatenate([x1_ref[...][:, :64], x2_ref[...][:, :64],
                           x3_ref[...][:, :64], x4], axis=1)
    g = jnp.dot(_bf(cat), _bf(w_ref[...]), preferred_element_type=jnp.float32)
    sg_ref[...] = jnp.sum(g, axis=0, keepdims=True)[None]
    sq_ref[...] = jnp.sum(g * g, axis=0, keepdims=True)[None]
    gm_ref[...] = jnp.max(g, axis=0, keepdims=True)[None, None]


def _conv_head(x1T, x2T, x3T, m4, g4, b4, mu4, den4, convWT):
    return pl.pallas_call(
        _conv_body,
        grid=(NBLK,),
        in_specs=[
            pl.BlockSpec((BLK, CP), lambda i: (i, 0)),
            pl.BlockSpec((BLK, CP), lambda i: (i, 0)),
            pl.BlockSpec((BLK, CP), lambda i: (i, 0)),
            pl.BlockSpec((BLK, 128), lambda i: (i, 0)),
            pl.BlockSpec((1, 128), lambda i: (0, 0)),
            pl.BlockSpec((1, 128), lambda i: (0, 0)),
            pl.BlockSpec((1, 128), lambda i: (0, 0)),
            pl.BlockSpec((1, 128), lambda i: (0, 0)),
            pl.BlockSpec((320, 1024), lambda i: (0, 0)),
        ],
        out_specs=[
            pl.BlockSpec((BLK, 128), lambda i: (i, 0)),
            pl.BlockSpec((1, 1, 1024), lambda i: (i, 0, 0)),
            pl.BlockSpec((1, 1, 1024), lambda i: (i, 0, 0)),
            pl.BlockSpec((1, 1, 1, 1024), lambda i: (i // 8, i % 8, 0, 0)),
        ],
        out_shape=[
            jax.ShapeDtypeStruct((ROWS, 128), jnp.float32),
            jax.ShapeDtypeStruct((NBLK, 1, 1024), jnp.float32),
            jax.ShapeDtypeStruct((NBLK, 1, 1024), jnp.float32),
            jax.ShapeDtypeStruct((B, 8, 1, 1024), jnp.float32),
        ],
    )(x1T, x2T, x3T, m4, g4.reshape(1, 128), b4.reshape(1, 128),
      mu4, den4, convWT)


def _gmax_body(sg_ref, sq_ref, gm_ref, g_ref, b_ref, w2t_ref, vb_ref):
    cnt = jnp.float32(ROWS)
    mu = jnp.sum(sg_ref[:, 0, :], axis=0, keepdims=True) / cnt
    var = jnp.sum(sq_ref[:, 0, :], axis=0, keepdims=True) / cnt - mu * mu
    den = jnp.sqrt(var + EPS)
    rows = [jnp.max(gm_ref[b, :, 0, :], axis=0, keepdims=True)
            for b in range(B)]
    gm = jnp.concatenate(rows, axis=0)
    feat = _lrelu(_bn_apply(gm, g_ref[...], b_ref[...], mu, den))
    vb = jnp.dot(_bf(feat), _bf(w2t_ref[...]),
                 preferred_element_type=jnp.float32)
    vb_ref[...] = vb[:, None, :]


def _gmax_head(sg, sq, gmax, conv_g, conv_b, seg1W2T):
    return pl.pallas_call(
        _gmax_body,
        in_specs=[pl.BlockSpec((NBLK, 1, 1024), lambda: (0, 0, 0)),
                  pl.BlockSpec((NBLK, 1, 1024), lambda: (0, 0, 0)),
                  pl.BlockSpec((B, 8, 1, 1024), lambda: (0, 0, 0, 0)),
                  pl.BlockSpec((1, 1024), lambda: (0, 0)),
                  pl.BlockSpec((1, 1024), lambda: (0, 0)),
                  pl.BlockSpec((1024, 512), lambda: (0, 0))],
        out_specs=pl.BlockSpec((B, 1, 512), lambda: (0, 0, 0)),
        out_shape=jax.ShapeDtypeStruct((B, 1, 512), jnp.float32),
    )(sg, sq, gmax, conv_g.reshape(1, 1024), conv_b.reshape(1, 1024), seg1W2T)


def _seg1_body(x1_ref, x2_ref, x3_ref, x4_ref, w_ref, vb_ref,
               h_ref, sp_ref, qp_ref):
    cat = jnp.concatenate([x1_ref[...][:, :64], x2_ref[...][:, :64],
                           x3_ref[...][:, :64], x4_ref[...]], axis=1)
    h = jnp.dot(_bf(cat), _bf(w_ref[...]),
                preferred_element_type=jnp.float32) + vb_ref[0]
    h_ref[...] = h
    sp_ref[...] = jnp.sum(h, axis=0, keepdims=True)[None]
    qp_ref[...] = jnp.sum(h * h, axis=0, keepdims=True)[None]


def _seg1(x1T, x2T, x3T, x4T, seg1W1T, vbias):
    return pl.pallas_call(
        _seg1_body,
        grid=(NBLK,),
        in_specs=[
            pl.BlockSpec((BLK, CP), lambda i: (i, 0)),
            pl.BlockSpec((BLK, CP), lambda i: (i, 0)),
            pl.BlockSpec((BLK, CP), lambda i: (i, 0)),
            pl.BlockSpec((BLK, 128), lambda i: (i, 0)),
            pl.BlockSpec((320, 512), lambda i: (0, 0)),
            pl.BlockSpec((1, 1, 512), lambda i: (i // 8, 0, 0)),
        ],
        out_specs=[
            pl.BlockSpec((BLK, 512), lambda i: (i, 0)),
            pl.BlockSpec((1, 1, 512), lambda i: (i, 0, 0)),
            pl.BlockSpec((1, 1, 512), lambda i: (i, 0, 0)),
        ],
        out_shape=[
            jax.ShapeDtypeStruct((ROWS, 512), jnp.float32),
            jax.ShapeDtypeStruct((NBLK, 1, 512), jnp.float32),
            jax.ShapeDtypeStruct((NBLK, 1, 512), jnp.float32),
        ],
    )(x1T, x2T, x3T, x4T, seg1W1T, vbias)


def _seg_body(h_ref, g_ref, b_ref, mu_ref, den_ref, w_ref, bias_ref,
              o_ref, sp_ref, qp_ref):
    t = _lrelu(_bn_apply(h_ref[...], g_ref[...], b_ref[...],
                         mu_ref[...], den_ref[...]))
    o = jnp.dot(_bf(t), _bf(w_ref[...]),
                preferred_element_type=jnp.float32) + bias_ref[...]
    o_ref[...] = o
    if sp_ref is not None:
        sp_ref[...] = jnp.sum(o, axis=0, keepdims=True)[None]
        qp_ref[...] = jnp.sum(o * o, axis=0, keepdims=True)[None]


def _seg(h, g, b, mu, den, wT, bias, with_stats):
    cin = h.shape[1]
    cout = wT.shape[1]
    body = _seg_body if with_stats else (
        lambda h_ref, g_ref, b_ref, mu_ref, den_ref, w_ref, bias_ref, o_ref:
        _seg_body(h_ref, g_ref, b_ref, mu_ref, den_ref, w_ref, bias_ref,
                  o_ref, None, None))
    out_specs = [pl.BlockSpec((BLK, cout), lambda i: (i, 0))]
    out_shape = [jax.ShapeDtypeStruct((ROWS, cout), jnp.float32)]
    if with_stats:
        out_specs += [pl.BlockSpec((1, 1, cout), lambda i: (i, 0, 0)),
                      pl.BlockSpec((1, 1, cout), lambda i: (i, 0, 0))]
        out_shape += [jax.ShapeDtypeStruct((NBLK, 1, cout), jnp.float32),
                      jax.ShapeDtypeStruct((NBLK, 1, cout), jnp.float32)]
    return pl.pallas_call(
        body,
        grid=(NBLK,),
        in_specs=[
            pl.BlockSpec((BLK, cin), lambda i: (i, 0)),
            pl.BlockSpec((1, cin), lambda i: (0, 0)),
            pl.BlockSpec((1, cin), lambda i: (0, 0)),
            pl.BlockSpec((1, cin), lambda i: (0, 0)),
            pl.BlockSpec((1, cin), lambda i: (0, 0)),
            pl.BlockSpec((cin, cout), lambda i: (0, 0)),
            pl.BlockSpec((1, cout), lambda i: (0, 0)),
        ],
        out_specs=out_specs,
        out_shape=out_shape,
    )(h, g.reshape(1, cin), b.reshape(1, cin), mu, den, wT, bias)


def _shadow_stats(E, w, ce, cin):
    e4 = E.reshape(B, N, K, 2 * ce)
    if ce != cin:
        e4 = jnp.concatenate([e4[..., :cin], e4[..., ce:ce + cin]], axis=-1)
    edge = jax.lax.optimization_barrier(jnp.transpose(e4, (0, 3, 1, 2)))
    h = jnp.einsum('oc,bcnk->bonk', w, edge)
    mu = jnp.mean(h, axis=(0, 2, 3), keepdims=True)
    var = jnp.mean((h - mu) ** 2, axis=(0, 2, 3), keepdims=True)
    o = w.shape[0]
    return mu.reshape(1, o), jnp.sqrt(var + EPS).reshape(1, o)


def _edge_layer(xTp, xxrow, w, cin, ce):
    w1t = w[:, :cin].T
    w2t = w[:, cin:].T
    if ce != cin:
        w1t = jnp.pad(w1t, ((0, ce - cin), (0, 0)))
        w2t = jnp.pad(w2t, ((0, ce - cin), (0, 0)))
    wt = jnp.concatenate([w1t, w2t], axis=0)
    idx = _knn(xTp, xxrow)
    E = _sc_edges(xTp, idx, ce)
    m = _edgered(E, wt)
    mu, den = _shadow_stats(E, w, ce, cin)
    return m, mu, den


def kernel(x, params):
    p = params
    x0 = x.reshape(ROWS, 3)
    x0p = jnp.pad(x0, ((0, 0), (0, CP - 3)))
    xx1 = jnp.sum(jnp.transpose(x, (0, 2, 1)) ** 2, axis=1, keepdims=True)

    m1, mu1, den1 = _edge_layer(x0p, xx1, p['ec1_W'], 3, 16)
    x1p, xsq1 = _apply(m1, p['ec1_g'], p['ec1_b'], mu1, den1)

    m2, mu2, den2 = _edge_layer(x1p, xsq1.reshape(B, 1, N), p['ec2_W'], 64, 64)
    x2p, xsq2 = _apply(m2, p['ec2_g'], p['ec2_b'], mu2, den2)

    m3, mu3, den3 = _edge_layer(x2p, xsq2.reshape(B, 1, N), p['ec3_W'], 64, 64)
    x3p, xsq3 = _apply(m3, p['ec3_g'], p['ec3_b'], mu3, den3)

    m4, mu4, den4 = _edge_layer(x3p, xsq3.reshape(B, 1, N), p['ec4_W'], 64, 64)

    x4T, sg, sq, gmax = _conv_head(x1p, x2p, x3p, m4, p['ec4_g'], p['ec4_b'],
                                   mu4, den4, p['conv_W'].T)
    vbias = _gmax_head(sg, sq, gmax, p['conv_g'], p['conv_b'],
                       p['seg1_W'][:, 320:].T)
    h1, sp1, qp1 = _seg1(x1p, x2p, x3p, x4T, p['seg1_W'][:, :320].T, vbias)
    mu_s1, den_s1 = _pstats(sp1, qp1, ROWS)
    zb256 = jnp.zeros((1, 256), jnp.float32)
    h2, sp2, qp2 = _seg(h1, p['seg1_g'], p['seg1_b'], mu_s1, den_s1,
                        p['seg2_W'].T, zb256, True)
    mu_s2, den_s2 = _pstats(sp2, qp2, ROWS)
    w3t = jnp.pad(p['seg3_W'].T, ((0, 0), (0, 14)))
    b3 = jnp.pad(p['seg3_bias'].reshape(1, 50), ((0, 0), (0, 14)))
    (out,) = _seg(h2, p['seg2_g'], p['seg2_b'], mu_s2, den_s2, w3t, b3, False)
    return out[:, :50].reshape(B, N, 50)


# speedup vs baseline: 5.3642x; 1.0049x over previous
import functools

import jax
import jax.numpy as jnp
from jax import lax
from jax.experimental import pallas as pl
from jax.experimental.pallas import tpu as pltpu
from jax.experimental.pallas import tpu_sc as plsc

B = 4
N = 2048
K = 20
KPAD = 128
CP = 128
ROWS = B * N
BLK = 256
NBLK = ROWS // BLK
EPS = 1e-5

NC = 2
NS = 16
NW = NC * NS
PPW = ROWS // NW
CHP = 8
CHROWS = CHP * K
NCH = PPW // CHP
IDXCH = 64


def _lrelu(v):
    return jnp.where(v >= 0.0, v, 0.2 * v)


def _bf(v):
    return v.astype(jnp.bfloat16)


KBLK = 512


def _knn_body(blk_ref, slab_ref, xx_ref, idx_ref, dist_ref):
    b = pl.program_id(0)
    blk = blk_ref[...]
    slab = slab_ref[...]
    inner = lax.dot_general(_bf(blk), _bf(slab), (((1,), (1,)), ((), ())),
                            preferred_element_type=jnp.float32)
    xxb = jnp.sum(blk * blk, axis=1, keepdims=True)
    xxs = xx_ref[0]
    dist_ref[...] = (xxb + xxs) - 2.0 * inner
    iota = lax.broadcasted_iota(jnp.int32, (KBLK, N), 1)
    base = b * N
    cols = []
    for _ in range(K):
        d = dist_ref[...]
        rowmin = jnp.min(d, axis=1, keepdims=True)
        cand = jnp.where(d == rowmin, iota, jnp.int32(2 * N))
        sel = jnp.min(cand, axis=1, keepdims=True)
        cols.append(sel + base)
        dist_ref[...] = jnp.where(iota == sel, jnp.float32(jnp.inf), d)
    cols.append(jnp.zeros((KBLK, KPAD - K), jnp.int32))
    idx_ref[...] = jnp.concatenate(cols, axis=1)


def _knn(xT, xxrow):
    return pl.pallas_call(
        _knn_body,
        grid=(B, N // KBLK),
        in_specs=[
            pl.BlockSpec((KBLK, CP), lambda b, i: (b * (N // KBLK) + i, 0)),
            pl.BlockSpec((N, CP), lambda b, i: (b, 0)),
            pl.BlockSpec((1, 1, N), lambda b, i: (b, 0, 0)),
        ],
        out_specs=pl.BlockSpec((KBLK, KPAD),
                               lambda b, i: (b * (N // KBLK) + i, 0)),
        out_shape=jax.ShapeDtypeStruct((ROWS, KPAD), jnp.int32),
        scratch_shapes=[pltpu.VMEM((KBLK, N), jnp.float32)],
    )(xT, xT, xxrow)


def _sc_body(ce, x_hbm, idx_hbm, e_hbm,
             idx_v, lst_v, buf_v, xi_v, e_v, sem0, sem1, osem0, osem1):
    wid = lax.axis_index("s") * NC + lax.axis_index("c")
    base = wid * PPW

    lane = lax.iota(jnp.int32, NS)

    for t in range(PPW // IDXCH):
        pltpu.sync_copy(idx_hbm.at[pl.ds(base + t * IDXCH, IDXCH)], idx_v)

        def build(pp, carry, t=t):
            for half in range(2):
                v = idx_v[pp, pl.ds(half * NS, NS)]
                pos = (t * IDXCH + pp) * K + half * NS + lane
                msk = (half * NS + lane) < K
                plsc.store_scatter(lst_v, (pos,), v, mask=msk)
            return carry

        lax.fori_loop(0, IDXCH, build, 0)

    sems = (sem0, sem1)
    osems = (osem0, osem1)
    ew = K * ce

    def fire(g, slot):
        pltpu.async_copy(
            x_hbm.at[lst_v.at[pl.ds(g * CHROWS, CHROWS)]],
            buf_v.at[slot], sems[slot])

    fire(0, 0)
    fire(1, 1)

    def chunk(g, slot):
        pltpu.make_async_copy(
            x_hbm.at[lst_v.at[pl.ds(g * CHROWS, CHROWS)]],
            buf_v.at[slot], sems[slot]).wait()
        pltpu.sync_copy(x_hbm.at[pl.ds(base + g * CHP, CHP)], xi_v)

        @pl.when(g >= 2)
        def _():
            pltpu.make_async_copy(e_v.at[slot],
                                  e_hbm.at[pl.ds(base + (g - 2) * CHP, CHP)],
                                  osems[slot]).wait()

        def point(p, carry):
            row = p * K
            for c in range(ce // NS):
                cs = pl.ds(c * NS, NS)
                xi = xi_v[p, cs]
                for j in range(K):
                    e_v[slot, p, pl.ds(j * ce + c * NS, NS)] = (
                        buf_v[slot, row + j, cs] - xi)
            return carry

        lax.fori_loop(0, CHP, point, 0)

        pltpu.async_copy(e_v.at[slot],
                         e_hbm.at[pl.ds(base + g * CHP, CHP)], osems[slot])

        @pl.when(g + 2 < NCH)
        def _():
            fire(g + 2, slot)

    def pair(t, carry):
        chunk(2 * t, 0)
        chunk(2 * t + 1, 1)
        return carry

    lax.fori_loop(0, NCH // 2, pair, 0)

    pltpu.make_async_copy(e_v.at[0],
                          e_hbm.at[pl.ds(base + (NCH - 2) * CHP, CHP)],
                          osems[0]).wait()
    pltpu.make_async_copy(e_v.at[1],
                          e_hbm.at[pl.ds(base + (NCH - 1) * CHP, CHP)],
                          osems[1]).wait()


def _sc_edges(xT, idx, ce):
    mesh = plsc.VectorSubcoreMesh(core_axis_name="c", subcore_axis_name="s",
                                  num_cores=NC, num_subcores=NS)
    fn = pl.kernel(
        functools.partial(_sc_body, ce),
        out_type=jax.ShapeDtypeStruct((ROWS, K * ce), jnp.float32),
        mesh=mesh,
        compiler_params=pltpu.CompilerParams(needs_layout_passes=False),
        scratch_types=[
            pltpu.VMEM((IDXCH, KPAD), jnp.int32),
            pltpu.VMEM((PPW * K,), jnp.int32),
            pltpu.VMEM((2, CHROWS, CP), jnp.float32),
            pltpu.VMEM((CHP, CP), jnp.float32),
            pltpu.VMEM((2, CHP, K * ce), jnp.float32),
            pltpu.SemaphoreType.DMA,
            pltpu.SemaphoreType.DMA,
            pltpu.SemaphoreType.DMA,
            pltpu.SemaphoreType.DMA,
        ],
    )
    return fn(xT, idx)


def _ered_body(ce, e_ref, x_ref, wt_ref, m_ref):
    w = _bf(wt_ref[...])
    e = _bf(e_ref[...])
    xi = _bf(x_ref[...][:, :ce])
    mx = None
    for j in range(K):
        op = jnp.concatenate([xi, e[:, j * ce:(j + 1) * ce]], axis=1)
        hj = jnp.dot(op, w, preferred_element_type=jnp.float32)
        mx = hj if mx is None else jnp.maximum(mx, hj)
    m_ref[...] = mx


def _edgered(E, xT, wt):
    w2c = wt.shape[0]
    ce = w2c // 2
    o = wt.shape[1]
    return pl.pallas_call(
        functools.partial(_ered_body, ce),
        grid=(NBLK,),
        in_specs=[
            pl.BlockSpec((BLK, K * ce), lambda i: (i, 0)),
            pl.BlockSpec((BLK, CP), lambda i: (i, 0)),
            pl.BlockSpec((w2c, o), lambda i: (0, 0)),
        ],
        out_specs=pl.BlockSpec((BLK, o), lambda i: (i, 0)),
        out_shape=jax.ShapeDtypeStruct((ROWS, o), jnp.float32),
    )(E, xT, wt)


def _pstats_body(cnt, sp_ref, qp_ref, mu_ref, den_ref):
    mu = jnp.sum(sp_ref[:, 0, :], axis=0, keepdims=True) / cnt
    var = jnp.sum(qp_ref[:, 0, :], axis=0, keepdims=True) / cnt - mu * mu
    mu_ref[...] = mu
    den_ref[...] = jnp.sqrt(var + EPS)


def _pstats(sp, qp, cnt):
    c = sp.shape[2]
    return pl.pallas_call(
        functools.partial(_pstats_body, float(cnt)),
        in_specs=[pl.BlockSpec((NBLK, 1, c), lambda: (0, 0, 0)),
                  pl.BlockSpec((NBLK, 1, c), lambda: (0, 0, 0))],
        out_specs=[pl.BlockSpec((1, c), lambda: (0, 0)),
                   pl.BlockSpec((1, c), lambda: (0, 0))],
        out_shape=[jax.ShapeDtypeStruct((1, c), jnp.float32),
                   jax.ShapeDtypeStruct((1, c), jnp.float32)],
    )(sp, qp)


def _bn_apply(v, g, b, mu, den):
    return g * (v - mu) / den + b


def _apply_body(m_ref, g_ref, b_ref, mu_ref, den_ref, x_ref, xsq_ref):
    v = _lrelu(_bn_apply(m_ref[...], g_ref[...], b_ref[...],
                         mu_ref[...], den_ref[...]))
    x_ref[...] = jnp.concatenate(
        [v, jnp.zeros((BLK, CP - v.shape[1]), jnp.float32)], axis=1)
    xsq_ref[...] = jnp.sum(v * v, axis=1, keepdims=True)


def _apply(m, g, b, mu, den):
    c = m.shape[1]
    return pl.pallas_call(
        _apply_body,
        grid=(NBLK,),
        in_specs=[
            pl.BlockSpec((BLK, c), lambda i: (i, 0)),
            pl.BlockSpec((1, c), lambda i: (0, 0)),
            pl.BlockSpec((1, c), lambda i: (0, 0)),
            pl.BlockSpec((1, c), lambda i: (0, 0)),
            pl.BlockSpec((1, c), lambda i: (0, 0)),
        ],
        out_specs=[
            pl.BlockSpec((BLK, CP), lambda i: (i, 0)),
            pl.BlockSpec((BLK, 1), lambda i: (i, 0)),
        ],
        out_shape=[
            jax.ShapeDtypeStruct((ROWS, CP), jnp.float32),
            jax.ShapeDtypeStruct((ROWS, 1), jnp.float32),
        ],
    )(m, g.reshape(1, c), b.reshape(1, c), mu, den)


def _conv_body(x1_ref, x2_ref, x3_ref, m4_ref, g4_ref, b4_ref, mu_ref,
               den_ref, w_ref, x4_ref, sg_ref, sq_ref, gm_ref):
    x4 = _lrelu(_bn_apply(m4_ref[...], g4_ref[...], b4_ref[...],
                          mu_ref[...], den_ref[...]))
    x4_ref[...] = x4
    cat = jnp.concatenate([x1_ref[...][:, :64], x2_ref[...][:, :64],
                           x3_ref[...][:, :64], x4], axis=1)
    g = jnp.dot(_bf(cat), _bf(w_ref[...]), preferred_element_type=jnp.float32)
    sg_ref[...] = jnp.sum(g, axis=0, keepdims=True)[None]
    sq_ref[...] = jnp.sum(g * g, axis=0, keepdims=True)[None]
    gm_ref[...] = jnp.max(g, axis=0, keepdims=True)[None, None]


def _conv_head(x1T, x2T, x3T, m4, g4, b4, mu4, den4, convWT):
    return pl.pallas_call(
        _conv_body,
        grid=(NBLK,),
        in_specs=[
            pl.BlockSpec((BLK, CP), lambda i: (i, 0)),
            pl.BlockSpec((BLK, CP), lambda i: (i, 0)),
            pl.BlockSpec((BLK, CP), lambda i: (i, 0)),
            pl.BlockSpec((BLK, 128), lambda i: (i, 0)),
            pl.BlockSpec((1, 128), lambda i: (0, 0)),
            pl.BlockSpec((1, 128), lambda i: (0, 0)),
            pl.BlockSpec((1, 128), lambda i: (0, 0)),
            pl.BlockSpec((1, 128), lambda i: (0, 0)),
            pl.BlockSpec((320, 1024), lambda i: (0, 0)),
        ],
        out_specs=[
            pl.BlockSpec((BLK, 128), lambda i: (i, 0)),
            pl.BlockSpec((1, 1, 1024), lambda i: (i, 0, 0)),
            pl.BlockSpec((1, 1, 1024), lambda i: (i, 0, 0)),
            pl.BlockSpec((1, 1, 1, 1024), lambda i: (i // 8, i % 8, 0, 0)),
        ],
        out_shape=[
            jax.ShapeDtypeStruct((ROWS, 128), jnp.float32),
            jax.ShapeDtypeStruct((NBLK, 1, 1024), jnp.float32),
            jax.ShapeDtypeStruct((NBLK, 1, 1024), jnp.float32),
            jax.ShapeDtypeStruct((B, 8, 1, 1024), jnp.float32),
        ],
    )(x1T, x2T, x3T, m4, g4.reshape(1, 128), b4.reshape(1, 128),
      mu4, den4, convWT)


def _gmax_body(sg_ref, sq_ref, gm_ref, g_ref, b_ref, w2t_ref, vb_ref):
    cnt = jnp.float32(ROWS)
    mu = jnp.sum(sg_ref[:, 0, :], axis=0, keepdims=True) / cnt
    var = jnp.sum(sq_ref[:, 0, :], axis=0, keepdims=True) / cnt - mu * mu
    den = jnp.sqrt(var + EPS)
    rows = [jnp.max(gm_ref[b, :, 0, :], axis=0, keepdims=True)
            for b in range(B)]
    gm = jnp.concatenate(rows, axis=0)
    feat = _lrelu(_bn_apply(gm, g_ref[...], b_ref[...], mu, den))
    vb = jnp.dot(_bf(feat), _bf(w2t_ref[...]),
                 preferred_element_type=jnp.float32)
    vb_ref[...] = vb[:, None, :]


def _gmax_head(sg, sq, gmax, conv_g, conv_b, seg1W2T):
    return pl.pallas_call(
        _gmax_body,
        in_specs=[pl.BlockSpec((NBLK, 1, 1024), lambda: (0, 0, 0)),
                  pl.BlockSpec((NBLK, 1, 1024), lambda: (0, 0, 0)),
                  pl.BlockSpec((B, 8, 1, 1024), lambda: (0, 0, 0, 0)),
                  pl.BlockSpec((1, 1024), lambda: (0, 0)),
                  pl.BlockSpec((1, 1024), lambda: (0, 0)),
                  pl.BlockSpec((1024, 512), lambda: (0, 0))],
        out_specs=pl.BlockSpec((B, 1, 512), lambda: (0, 0, 0)),
        out_shape=jax.ShapeDtypeStruct((B, 1, 512), jnp.float32),
    )(sg, sq, gmax, conv_g.reshape(1, 1024), conv_b.reshape(1, 1024), seg1W2T)


def _seg1_body(x1_ref, x2_ref, x3_ref, x4_ref, w_ref, vb_ref,
               h_ref, sp_ref, qp_ref):
    cat = jnp.concatenate([x1_ref[...][:, :64], x2_ref[...][:, :64],
                           x3_ref[...][:, :64], x4_ref[...]], axis=1)
    h = jnp.dot(_bf(cat), _bf(w_ref[...]),
                preferred_element_type=jnp.float32) + vb_ref[0]
    h_ref[...] = h
    sp_ref[...] = jnp.sum(h, axis=0, keepdims=True)[None]
    qp_ref[...] = jnp.sum(h * h, axis=0, keepdims=True)[None]


def _seg1(x1T, x2T, x3T, x4T, seg1W1T, vbias):
    return pl.pallas_call(
        _seg1_body,
        grid=(NBLK,),
        in_specs=[
            pl.BlockSpec((BLK, CP), lambda i: (i, 0)),
            pl.BlockSpec((BLK, CP), lambda i: (i, 0)),
            pl.BlockSpec((BLK, CP), lambda i: (i, 0)),
            pl.BlockSpec((BLK, 128), lambda i: (i, 0)),
            pl.BlockSpec((320, 512), lambda i: (0, 0)),
            pl.BlockSpec((1, 1, 512), lambda i: (i // 8, 0, 0)),
        ],
        out_specs=[
            pl.BlockSpec((BLK, 512), lambda i: (i, 0)),
            pl.BlockSpec((1, 1, 512), lambda i: (i, 0, 0)),
            pl.BlockSpec((1, 1, 512), lambda i: (i, 0, 0)),
        ],
        out_shape=[
            jax.ShapeDtypeStruct((ROWS, 512), jnp.float32),
            jax.ShapeDtypeStruct((NBLK, 1, 512), jnp.float32),
            jax.ShapeDtypeStruct((NBLK, 1, 512), jnp.float32),
        ],
    )(x1T, x2T, x3T, x4T, seg1W1T, vbias)


def _seg_body(h_ref, g_ref, b_ref, mu_ref, den_ref, w_ref, bias_ref,
              o_ref, sp_ref, qp_ref):
    t = _lrelu(_bn_apply(h_ref[...], g_ref[...], b_ref[...],
                         mu_ref[...], den_ref[...]))
    o = jnp.dot(_bf(t), _bf(w_ref[...]),
                preferred_element_type=jnp.float32) + bias_ref[...]
    o_ref[...] = o
    if sp_ref is not None:
        sp_ref[...] = jnp.sum(o, axis=0, keepdims=True)[None]
        qp_ref[...] = jnp.sum(o * o, axis=0, keepdims=True)[None]


def _seg(h, g, b, mu, den, wT, bias, with_stats):
    cin = h.shape[1]
    cout = wT.shape[1]
    body = _seg_body if with_stats else (
        lambda h_ref, g_ref, b_ref, mu_ref, den_ref, w_ref, bias_ref, o_ref:
        _seg_body(h_ref, g_ref, b_ref, mu_ref, den_ref, w_ref, bias_ref,
                  o_ref, None, None))
    out_specs = [pl.BlockSpec((BLK, cout), lambda i: (i, 0))]
    out_shape = [jax.ShapeDtypeStruct((ROWS, cout), jnp.float32)]
    if with_stats:
        out_specs += [pl.BlockSpec((1, 1, cout), lambda i: (i, 0, 0)),
                      pl.BlockSpec((1, 1, cout), lambda i: (i, 0, 0))]
        out_shape += [jax.ShapeDtypeStruct((NBLK, 1, cout), jnp.float32),
                      jax.ShapeDtypeStruct((NBLK, 1, cout), jnp.float32)]
    return pl.pallas_call(
        body,
        grid=(NBLK,),
        in_specs=[
            pl.BlockSpec((BLK, cin), lambda i: (i, 0)),
            pl.BlockSpec((1, cin), lambda i: (0, 0)),
            pl.BlockSpec((1, cin), lambda i: (0, 0)),
            pl.BlockSpec((1, cin), lambda i: (0, 0)),
            pl.BlockSpec((1, cin), lambda i: (0, 0)),
            pl.BlockSpec((cin, cout), lambda i: (0, 0)),
            pl.BlockSpec((1, cout), lambda i: (0, 0)),
        ],
        out_specs=out_specs,
        out_shape=out_shape,
    )(h, g.reshape(1, cin), b.reshape(1, cin), mu, den, wT, bias)


def _shadow_stats(E, xTp, w, ce, cin):
    e4 = E.reshape(B, N, K, ce)[..., :cin]
    xi4 = jnp.broadcast_to(xTp[:, :cin].reshape(B, N, 1, cin), e4.shape)
    cat = jnp.concatenate([xi4, e4], axis=-1)
    edge = jax.lax.optimization_barrier(jnp.transpose(cat, (0, 3, 1, 2)))
    h = jnp.einsum('oc,bcnk->bonk', w, edge)
    mu = jnp.mean(h, axis=(0, 2, 3), keepdims=True)
    var = jnp.mean((h - mu) ** 2, axis=(0, 2, 3), keepdims=True)
    o = w.shape[0]
    return mu.reshape(1, o), jnp.sqrt(var + EPS).reshape(1, o)


def _edge_layer(xTp, xxrow, w, cin, ce):
    w1t = w[:, :cin].T
    w2t = w[:, cin:].T
    if ce != cin:
        w1t = jnp.pad(w1t, ((0, ce - cin), (0, 0)))
        w2t = jnp.pad(w2t, ((0, ce - cin), (0, 0)))
    wt = jnp.concatenate([w1t, w2t], axis=0)
    idx = _knn(xTp, xxrow)
    E = _sc_edges(xTp, idx, ce)
    m = _edgered(E, xTp, wt)
    mu, den = _shadow_stats(E, xTp, w, ce, cin)
    return m, mu, den


def kernel(x, params):
    p = params
    x0 = x.reshape(ROWS, 3)
    x0p = jnp.pad(x0, ((0, 0), (0, CP - 3)))
    xx1 = jnp.sum(jnp.transpose(x, (0, 2, 1)) ** 2, axis=1, keepdims=True)

    m1, mu1, den1 = _edge_layer(x0p, xx1, p['ec1_W'], 3, 16)
    x1p, xsq1 = _apply(m1, p['ec1_g'], p['ec1_b'], mu1, den1)

    m2, mu2, den2 = _edge_layer(x1p, xsq1.reshape(B, 1, N), p['ec2_W'], 64, 64)
    x2p, xsq2 = _apply(m2, p['ec2_g'], p['ec2_b'], mu2, den2)

    m3, mu3, den3 = _edge_layer(x2p, xsq2.reshape(B, 1, N), p['ec3_W'], 64, 64)
    x3p, xsq3 = _apply(m3, p['ec3_g'], p['ec3_b'], mu3, den3)

    m4, mu4, den4 = _edge_layer(x3p, xsq3.reshape(B, 1, N), p['ec4_W'], 64, 64)

    x4T, sg, sq, gmax = _conv_head(x1p, x2p, x3p, m4, p['ec4_g'], p['ec4_b'],
                                   mu4, den4, p['conv_W'].T)
    vbias = _gmax_head(sg, sq, gmax, p['conv_g'], p['conv_b'],
                       p['seg1_W'][:, 320:].T)
    h1, sp1, qp1 = _seg1(x1p, x2p, x3p, x4T, p['seg1_W'][:, :320].T, vbias)
    mu_s1, den_s1 = _pstats(sp1, qp1, ROWS)
    zb256 = jnp.zeros((1, 256), jnp.float32)
    h2, sp2, qp2 = _seg(h1, p['seg1_g'], p['seg1_b'], mu_s1, den_s1,
                        p['seg2_W'].T, zb256, True)
    mu_s2, den_s2 = _pstats(sp2, qp2, ROWS)
    w3t = jnp.pad(p['seg3_W'].T, ((0, 0), (0, 14)))
    b3 = jnp.pad(p['seg3_bias'].reshape(1, 50), ((0, 0), (0, 14)))
    (out,) = _seg(h2, p['seg2_g'], p['seg2_b'], mu_s2, den_s2, w3t, b3, False)
    return out[:, :50].reshape(B, N, 50)
